# Initial kernel scaffold; baseline (speedup 1.0000x reference)
import jax
import jax.numpy as jnp
from jax.experimental import pallas as pl


def kernel(x):
    raise NotImplementedError("write your pallas kernel here")


# speedup vs baseline: 12.2448x; 12.2448x over previous
import jax
import jax.numpy as jnp
from jax.experimental import pallas as pl
from jax.experimental.pallas import tpu as pltpu


def _corner_pool_kernel(x_ref, o_ref):
    y = x_ref[...]
    neg = jnp.float32(-jnp.inf)
    d = 1
    while d < y.shape[1]:
        fill = jnp.full((y.shape[0], d, y.shape[2]), neg, y.dtype)
        y = jnp.maximum(y, jnp.concatenate([y[:, d:, :], fill], axis=1))
        d *= 2
    d = 1
    while d < y.shape[2]:
        fill = jnp.full((y.shape[0], y.shape[1], d), neg, y.dtype)
        y = jnp.maximum(y, jnp.concatenate([y[:, :, d:], fill], axis=2))
        d *= 2
    o_ref[...] = y + y


@jax.jit
def kernel(x):
    N, C, H, W = x.shape
    xr = x.reshape(N * C, H, W)
    B = 8
    grid = (N * C // B,)
    out = pl.pallas_call(
        _corner_pool_kernel,
        grid=grid,
        in_specs=[pl.BlockSpec((B, H, W), lambda i: (i, 0, 0))],
        out_specs=pl.BlockSpec((B, H, W), lambda i: (i, 0, 0)),
        out_shape=jax.ShapeDtypeStruct((N * C, H, W), x.dtype),
        compiler_params=pltpu.CompilerParams(
            dimension_semantics=("parallel",),
        ),
    )(xr)
    return out.reshape(N, C, H, W)


# speedup vs baseline: 16.1519x; 1.3191x over previous
import jax
import jax.numpy as jnp
from jax.experimental import pallas as pl
from jax.experimental.pallas import tpu as pltpu

_B = 8


def _sublane_suffix_max(y):
    neg = jnp.float32(-jnp.inf)
    d = 1
    while d < y.shape[1]:
        fill = jnp.full((y.shape[0], d, y.shape[2]), neg, y.dtype)
        y = jnp.maximum(y, jnp.concatenate([y[:, d:, :], fill], axis=1))
        d *= 2
    return y


def _corner_pool_kernel(x_ref, o_ref):
    y = _sublane_suffix_max(x_ref[...])
    y = jnp.swapaxes(y, 1, 2)
    y = _sublane_suffix_max(y)
    o_ref[...] = jnp.swapaxes(y + y, 1, 2)


@jax.jit
def kernel(x):
    N, C, H, W = x.shape
    xr = x.reshape(N * C, H, W)
    grid = (N * C // _B,)
    out = pl.pallas_call(
        _corner_pool_kernel,
        grid=grid,
        in_specs=[pl.BlockSpec((_B, H, W), lambda i: (i, 0, 0))],
        out_specs=pl.BlockSpec((_B, H, W), lambda i: (i, 0, 0)),
        out_shape=jax.ShapeDtypeStruct((N * C, H, W), x.dtype),
        compiler_params=pltpu.CompilerParams(
            dimension_semantics=("parallel",),
        ),
    )(xr)
    return out.reshape(N, C, H, W)
